# baseline (speedup 1.0000x reference)
import functools

import jax
import jax.numpy as jnp
from jax import lax
from jax.experimental import pallas as pl
from jax.experimental.pallas import tpu as pltpu
from jax.experimental.pallas import tpu_sc as plsc

_NW = 32
_R = 4


@functools.cache
def _build(n_rows: int, n_cols: int, d: int):
    assert n_rows % (_NW * _R) == 0
    rpw = n_rows // _NW
    nchunk = rpw // _R
    assert nchunk >= 4 and nchunk % 2 == 0

    mesh = plsc.VectorSubcoreMesh(core_axis_name="c", subcore_axis_name="s")

    @functools.partial(
        pl.kernel,
        out_type=jax.ShapeDtypeStruct((n_rows, n_cols, d), jnp.float32),
        mesh=mesh,
        scratch_types=[
            pltpu.VMEM((2, _R, n_cols), jnp.int32),
            pltpu.VMEM((2, _R, n_cols, d), jnp.float32),
            pltpu.SemaphoreType.DMA,
            pltpu.SemaphoreType.DMA,
        ],
        compiler_params=pltpu.CompilerParams(use_tc_tiling_on_sc=False),
    )
    def gather_kernel(idx_hbm, table_hbm, out_hbm, idx_v, rows_v, gsem, wsem):
        wid = lax.axis_index("s") * 2 + lax.axis_index("c")
        base = wid * rpw

        def fire_gather(i, b):
            row0 = base + i * _R
            pltpu.sync_copy(idx_hbm.at[pl.ds(row0, _R)], idx_v.at[b])
            for r in range(_R):
                pltpu.async_copy(
                    table_hbm.at[idx_v.at[b, r]], rows_v.at[b, r], gsem)

        def wait_gather(b):
            pltpu.make_async_copy(
                out_hbm.at[pl.ds(0, _R)], rows_v.at[b], gsem).wait()

        def fire_write(i, b):
            row0 = base + i * _R
            pltpu.async_copy(rows_v.at[b], out_hbm.at[pl.ds(row0, _R)], wsem)

        def wait_write(b):
            pltpu.make_async_copy(
                rows_v.at[b], out_hbm.at[pl.ds(0, _R)], wsem).wait()

        fire_gather(0, 0)
        fire_gather(1, 1)
        wait_gather(0)
        fire_write(0, 0)

        def pair_body(k, carry):
            i0 = 1 + 2 * k
            for p in range(2):
                i = i0 + p
                b = (1 + p) % 2
                nb = 1 - b
                wait_write(nb)
                fire_gather(i + 1, nb)
                wait_gather(b)
                fire_write(i, b)
            return carry

        lax.fori_loop(0, (nchunk - 2) // 2, pair_body, 0)

        b_last = (nchunk - 1) % 2
        wait_gather(b_last)
        fire_write(nchunk - 1, b_last)
        wait_write(0)
        wait_write(1)

    return gather_kernel


def kernel(indices, weight):
    n_rows, n_cols = indices.shape
    d = weight.shape[1]
    idx = indices.astype(jnp.int32)
    return _build(n_rows, n_cols, d)(idx, weight)


# speedup vs baseline: 1.3347x; 1.3347x over previous
import functools

import jax
import jax.numpy as jnp
from jax import lax
from jax.experimental import pallas as pl
from jax.experimental.pallas import tpu as pltpu
from jax.experimental.pallas import tpu_sc as plsc

_NW = 32
_R = 4


@functools.cache
def _build(n_rows: int, n_cols: int, d: int):
    assert n_rows % (_NW * _R) == 0
    rpw = n_rows // _NW
    nchunk = rpw // _R
    assert nchunk >= 4 and nchunk % 2 == 0

    mesh = plsc.VectorSubcoreMesh(core_axis_name="c", subcore_axis_name="s")

    @functools.partial(
        pl.kernel,
        out_type=jax.ShapeDtypeStruct((n_rows, n_cols, 2 * d), jnp.float32),
        mesh=mesh,
        scratch_types=[
            pltpu.VMEM((2, _R, n_cols), jnp.int32),
            pltpu.VMEM((2, _R, n_cols, d), jnp.float32),
            pltpu.SemaphoreType.DMA,
            pltpu.SemaphoreType.DMA,
        ],
        compiler_params=pltpu.CompilerParams(use_tc_tiling_on_sc=False),
    )
    def gather_kernel(idx_hbm, table_hbm, out_hbm, idx_v, rows_v, gsem, wsem):
        wid = lax.axis_index("s") * 2 + lax.axis_index("c")
        base = wid * rpw

        def fire_gather(i, b):
            row0 = base + i * _R
            pltpu.sync_copy(idx_hbm.at[pl.ds(row0, _R)], idx_v.at[b])
            for r in range(_R):
                pltpu.async_copy(
                    table_hbm.at[idx_v.at[b, r]], rows_v.at[b, r], gsem)

        def wait_gather(b):
            pltpu.make_async_copy(
                out_hbm.at[pl.ds(0, _R), :, pl.ds(0, d)],
                rows_v.at[b], gsem).wait()

        def fire_write(i, b):
            row0 = base + i * _R
            pltpu.async_copy(
                rows_v.at[b],
                out_hbm.at[pl.ds(row0, _R), :, pl.ds(0, d)], wsem)

        def wait_write(b):
            pltpu.make_async_copy(
                rows_v.at[b],
                out_hbm.at[pl.ds(0, _R), :, pl.ds(0, d)], wsem).wait()

        fire_gather(0, 0)
        fire_gather(1, 1)
        wait_gather(0)
        fire_write(0, 0)

        def pair_body(k, carry):
            i0 = 1 + 2 * k
            for p in range(2):
                i = i0 + p
                b = (1 + p) % 2
                nb = 1 - b
                wait_write(nb)
                fire_gather(i + 1, nb)
                wait_gather(b)
                fire_write(i, b)
            return carry

        lax.fori_loop(0, (nchunk - 2) // 2, pair_body, 0)

        b_last = (nchunk - 1) % 2
        wait_gather(b_last)
        fire_write(nchunk - 1, b_last)
        wait_write(0)
        wait_write(1)

    return gather_kernel


def kernel(indices, weight):
    n_rows, n_cols = indices.shape
    d = weight.shape[1]
    idx = indices.astype(jnp.int32)
    wide = _build(n_rows, n_cols, d)(idx, weight)
    return wide[:, :, :d]
